# Initial kernel scaffold; baseline (speedup 1.0000x reference)
import jax
import jax.numpy as jnp
from jax.experimental import pallas as pl


def kernel(x, task_id, te0, Ws0, bs0, W1_0, b1_0, g1_0, be1_0, W2_0, b2_0, g2_0, be2_0, W3_0, b3_0, te1, Ws1, bs1, W1_1, b1_1, g1_1, be1_1, W2_1, b2_1, g2_1, be2_1, W3_1, b3_1):
    raise NotImplementedError("write your pallas kernel here")


# speedup vs baseline: 1.0427x; 1.0427x over previous
import functools

import jax
import jax.numpy as jnp
from jax.experimental import pallas as pl

B, N, D = 4, 8192, 768
TASK_DIM = 64
PHASE_DIV = 2.0
BN = 2048


def _score_body(x_ref, temb0_ref, temb1_ref, w0_ref, w1_ref, b0_ref, b1_ref,
                s0_ref, s1_ref):
    xb = x_ref[0]
    t0 = jnp.broadcast_to(temb0_ref[0], (BN, TASK_DIM))
    t1 = jnp.broadcast_to(temb1_ref[0], (BN, TASK_DIM))
    h0 = jnp.concatenate([xb, t0], axis=-1)
    h1 = jnp.concatenate([xb, t1], axis=-1)
    s0 = h0 @ w0_ref[...] + b0_ref[...]
    s1 = h1 @ w1_ref[...] + b1_ref[...]
    s0_ref[0, 0, :] = s0[:, 0]
    s1_ref[0, 0, :] = s1[:, 0]


def _scores(x, temb0, temb1, Ws0, bs0, Ws1, bs1):
    grid = (B, N // BN)
    s0, s1 = pl.pallas_call(
        _score_body,
        grid=grid,
        in_specs=[
            pl.BlockSpec((1, BN, D), lambda b, n: (b, n, 0)),
            pl.BlockSpec((1, 1, TASK_DIM), lambda b, n: (b, 0, 0)),
            pl.BlockSpec((1, 1, TASK_DIM), lambda b, n: (b, 0, 0)),
            pl.BlockSpec((D + TASK_DIM, 1), lambda b, n: (0, 0)),
            pl.BlockSpec((D + TASK_DIM, 1), lambda b, n: (0, 0)),
            pl.BlockSpec((1,), lambda b, n: (0,)),
            pl.BlockSpec((1,), lambda b, n: (0,)),
        ],
        out_specs=[
            pl.BlockSpec((1, 1, BN), lambda b, n: (b, 0, n)),
            pl.BlockSpec((1, 1, BN), lambda b, n: (b, 0, n)),
        ],
        out_shape=[
            jax.ShapeDtypeStruct((B, 1, N), jnp.float32),
            jax.ShapeDtypeStruct((B, 1, N), jnp.float32),
        ],
    )(x, temb0.reshape(B, 1, TASK_DIM), temb1.reshape(B, 1, TASK_DIM),
      Ws0, Ws1, bs0, bs1)
    return s0[:, 0, :], s1[:, 0, :]


def _mlp_body(g_ref, sc_ref, temb_ref, W1_ref, b1_ref, g1_ref, be1_ref,
              W2_ref, b2_ref, g2_ref, be2_ref, W3_ref, b3_ref, out_ref, *, k):
    gr = g_ref[0]
    sc = sc_ref[0, 0]
    gated = gr * jax.nn.sigmoid(sc)[:, None]
    t = jnp.broadcast_to(temb_ref[0], (k, TASK_DIM))
    hk = jnp.concatenate([gated, t], axis=-1)

    z = hk @ W1_ref[...] + b1_ref[...]
    m = z.mean(-1, keepdims=True)
    v = z.var(-1, keepdims=True)
    z = (z - m) / jnp.sqrt(v + 1e-5) * g1_ref[...] + be1_ref[...]
    z = jax.nn.gelu(z)

    z = z @ W2_ref[...] + b2_ref[...]
    m = z.mean(-1, keepdims=True)
    v = z.var(-1, keepdims=True)
    z = (z - m) / jnp.sqrt(v + 1e-5) * g2_ref[...] + be2_ref[...]
    z = jax.nn.gelu(z)

    phase = jnp.sin((z @ W3_ref[...] + b3_ref[...]) * PHASE_DIV)
    out_ref[0] = jnp.concatenate([gated, phase], axis=-1)


def _mlp(gathered, topk_scores, temb, W1, b1, g1, be1, W2, b2, g2, be2, W3, b3, k):
    ph = W3.shape[-1]
    return pl.pallas_call(
        functools.partial(_mlp_body, k=k),
        grid=(B,),
        in_specs=[
            pl.BlockSpec((1, k, D), lambda b: (b, 0, 0)),
            pl.BlockSpec((1, 1, k), lambda b: (b, 0, 0)),
            pl.BlockSpec((1, 1, TASK_DIM), lambda b: (b, 0, 0)),
            pl.BlockSpec(W1.shape, lambda b: (0, 0)),
            pl.BlockSpec(b1.shape, lambda b: (0,)),
            pl.BlockSpec(g1.shape, lambda b: (0,)),
            pl.BlockSpec(be1.shape, lambda b: (0,)),
            pl.BlockSpec(W2.shape, lambda b: (0, 0)),
            pl.BlockSpec(b2.shape, lambda b: (0,)),
            pl.BlockSpec(g2.shape, lambda b: (0,)),
            pl.BlockSpec(be2.shape, lambda b: (0,)),
            pl.BlockSpec(W3.shape, lambda b: (0, 0)),
            pl.BlockSpec(b3.shape, lambda b: (0,)),
        ],
        out_specs=pl.BlockSpec((1, k, D + ph), lambda b: (b, 0, 0)),
        out_shape=jax.ShapeDtypeStruct((B, k, D + ph), jnp.float32),
    )(gathered, topk_scores.reshape(B, 1, k), temb.reshape(B, 1, TASK_DIM),
      W1, b1, g1, be1, W2, b2, g2, be2, W3, b3)


def kernel(x, task_id, te0, Ws0, bs0, W1_0, b1_0, g1_0, be1_0, W2_0, b2_0,
           g2_0, be2_0, W3_0, b3_0, te1, Ws1, bs1, W1_1, b1_1, g1_1, be1_1,
           W2_1, b2_1, g2_1, be2_1, W3_1, b3_1):
    temb0 = te0[task_id]
    temb1 = te1[task_id]
    s0, s1 = _scores(x, temb0, temb1, Ws0, bs0, Ws1, bs1)

    outs = []
    for (s, k, temb, W1, b1, g1, be1, W2, b2, g2, be2, W3, b3) in (
        (s0, 512, temb0, W1_0, b1_0, g1_0, be1_0, W2_0, b2_0, g2_0, be2_0, W3_0, b3_0),
        (s1, 256, temb1, W1_1, b1_1, g1_1, be1_1, W2_1, b2_1, g2_1, be2_1, W3_1, b3_1),
    ):
        topk_scores, topk_idx = jax.lax.top_k(s, k)
        gathered = jnp.take_along_axis(x, topk_idx[..., None], axis=1)
        outs.append(_mlp(gathered, topk_scores, temb, W1, b1, g1, be1,
                         W2, b2, g2, be2, W3, b3, k))
    return tuple(outs)
